# Initial kernel scaffold; baseline (speedup 1.0000x reference)
import jax
import jax.numpy as jnp
from jax.experimental import pallas as pl


def kernel(x, embedding, fc_w, fc_b):
    raise NotImplementedError("write your pallas kernel here")


# speedup vs baseline: 12.4613x; 12.4613x over previous
import functools

import jax
import jax.numpy as jnp
from jax import lax
from jax.experimental import pallas as pl
from jax.experimental.pallas import tpu as pltpu
from jax.experimental.pallas import tpu_sc as plsc

V = 100000
D = 64
L = 200
LP = 256
B = 4096

NC = 2
NS = 16
NW = NC * NS
BPW = B // NW
VBLK = 2000


def _mm_body(emb_ref, wt_ref, out_ref):
    out_ref[...] = jnp.dot(emb_ref[...], wt_ref[...],
                           preferred_element_type=jnp.float32)


def _scores_matmul(embedding, wt):
    return pl.pallas_call(
        _mm_body,
        grid=(V // VBLK,),
        in_specs=[
            pl.BlockSpec((VBLK, D), lambda i: (i, 0)),
            pl.BlockSpec((D, LP), lambda i: (0, 0)),
        ],
        out_specs=pl.BlockSpec((VBLK, LP), lambda i: (i, 0)),
        out_shape=jax.ShapeDtypeStruct((V, LP), jnp.float32),
    )(embedding, wt)


@functools.cache
def _make_sc_gather_reduce():
  @functools.partial(
      pl.kernel,
      out_type=jax.ShapeDtypeStruct((B,), jnp.float32),
      mesh=plsc.VectorSubcoreMesh(core_axis_name="c", subcore_axis_name="s",
                                  num_cores=NC, num_subcores=NS),
      scratch_types=[
          pltpu.VMEM((L, BPW), jnp.int32),
          pltpu.VMEM((L * BPW,), jnp.int32),
          pltpu.VMEM((L * BPW,), jnp.float32),
          pltpu.VMEM((BPW,), jnp.float32),
          pltpu.VMEM((16,), jnp.float32),
          pltpu.SemaphoreType.DMA,
      ],
  )
  def _sc_gather_reduce(xt_hbm, scores_hbm, fcb_hbm, out_hbm,
                        xv, idx, vals, outv, fcbv, sem):
    wid = lax.axis_index("s") * NC + lax.axis_index("c")
    base = wid * BPW

    pltpu.sync_copy(xt_hbm.at[:, pl.ds(base, BPW)], xv)
    pltpu.sync_copy(fcb_hbm, fcbv)

    def build_idx(l, carry):
      for k in range(BPW // 16):
        idx[pl.ds(l * BPW + k * 16, 16)] = xv[l, pl.ds(k * 16, 16)] * LP + l
      return carry
    lax.fori_loop(0, L, build_idx, 0)

    pltpu.async_copy(scores_hbm.at[idx], vals, sem).wait()

    def reduce_l(l, accs):
      return tuple(a + vals[pl.ds(l * BPW + k * 16, 16)]
                   for k, a in enumerate(accs))
    accs = lax.fori_loop(
        0, L, reduce_l,
        tuple(jnp.zeros((16,), jnp.float32) for _ in range(BPW // 16)))

    bias = fcbv[...]
    for k in range(BPW // 16):
      z = accs[k] + bias
      outv[pl.ds(k * 16, 16)] = 1.0 / (1.0 + jnp.exp(-z))

    pltpu.sync_copy(outv, out_hbm.at[pl.ds(base, BPW)])

  return _sc_gather_reduce


def kernel(x, embedding, fc_w, fc_b):
    x = x.astype(jnp.int32)
    wt = jnp.pad(fc_w[:, 0].reshape(L, D).T, ((0, 0), (0, LP - L)))
    scores = _scores_matmul(embedding, wt)
    scores_flat = scores.reshape(V * LP)
    xt = x.T
    fcb16 = jnp.broadcast_to(fc_b[0], (16,)).astype(jnp.float32)
    out = _make_sc_gather_reduce()(xt, scores_flat, fcb16)
    return out.reshape(B, 1)


# speedup vs baseline: 18.1167x; 1.4538x over previous
import functools

import jax
import jax.numpy as jnp
from jax import lax
from jax.experimental import pallas as pl
from jax.experimental.pallas import tpu as pltpu
from jax.experimental.pallas import tpu_sc as plsc

V = 100000
D = 64
L = 200
LP = 256
B = 4096

NC = 2
NS = 16
NW = NC * NS
BPW = B // NW
VBLK = 2000


def _mm_body(emb_ref, wt_ref, out_ref):
    res = jnp.dot(emb_ref[...], wt_ref[...],
                  preferred_element_type=jnp.float32)
    out_ref[...] = res.reshape(2 * VBLK, 128)


def _scores_matmul(embedding, wt):
    return pl.pallas_call(
        _mm_body,
        grid=(V // VBLK,),
        in_specs=[
            pl.BlockSpec((VBLK, D), lambda i: (i, 0)),
            pl.BlockSpec((D, LP), lambda i: (0, 0)),
        ],
        out_specs=pl.BlockSpec((2 * VBLK, 128), lambda i: (i, 0)),
        out_shape=jax.ShapeDtypeStruct((2 * V, 128), jnp.float32),
    )(embedding, wt)


@functools.cache
def _make_sc_gather_reduce():
  @functools.partial(
      pl.kernel,
      out_type=jax.ShapeDtypeStruct((B,), jnp.float32),
      mesh=plsc.VectorSubcoreMesh(core_axis_name="c", subcore_axis_name="s",
                                  num_cores=NC, num_subcores=NS),
      scratch_types=[
          pltpu.VMEM((L, BPW), jnp.int32),
          pltpu.VMEM((L * BPW,), jnp.int32),
          pltpu.VMEM((L * BPW,), jnp.float32),
          pltpu.VMEM((BPW,), jnp.float32),
          pltpu.VMEM((16,), jnp.float32),
          pltpu.SemaphoreType.DMA,
      ],
  )
  def _sc_gather_reduce(xt_hbm, scores_hbm, fcb_hbm, out_hbm,
                        xv, idx, vals, outv, fcbv, sem):
    wid = lax.axis_index("s") * NC + lax.axis_index("c")
    base = wid * BPW

    pltpu.sync_copy(xt_hbm.at[:, pl.ds(base, BPW)], xv)
    pltpu.sync_copy(fcb_hbm, fcbv)

    def build_idx(l, carry):
      for k in range(BPW // 16):
        idx[pl.ds(l * BPW + k * 16, 16)] = xv[l, pl.ds(k * 16, 16)] * LP + l
      return carry
    lax.fori_loop(0, L, build_idx, 0)

    pltpu.async_copy(scores_hbm.at[idx], vals, sem).wait()

    def reduce_l(l, accs):
      return tuple(a + vals[pl.ds(l * BPW + k * 16, 16)]
                   for k, a in enumerate(accs))
    accs = lax.fori_loop(
        0, L, reduce_l,
        tuple(jnp.zeros((16,), jnp.float32) for _ in range(BPW // 16)))

    bias = fcbv[...]
    for k in range(BPW // 16):
      z = accs[k] + bias
      outv[pl.ds(k * 16, 16)] = 1.0 / (1.0 + jnp.exp(-z))

    pltpu.sync_copy(outv, out_hbm.at[pl.ds(base, BPW)])

  return _sc_gather_reduce


def kernel(x, embedding, fc_w, fc_b):
    x = x.astype(jnp.int32)
    wt = jnp.pad(fc_w[:, 0].reshape(L, D).T, ((0, 0), (0, LP - L)))
    scores = _scores_matmul(embedding, wt)
    scores_flat = scores.reshape(V * LP)
    xt = x.T
    fcb16 = jnp.broadcast_to(fc_b[0], (16,)).astype(jnp.float32)
    out = _make_sc_gather_reduce()(xt, scores_flat, fcb16)
    return out.reshape(B, 1)
